# baseline (speedup 1.0000x reference)
import jax
import jax.numpy as jnp
from jax.experimental import pallas as pl
from jax.experimental.pallas import tpu as pltpu

EMB_DIM = 512
N_CLASSES = 1139
N_TYPES = 4
BATCH = 16384

BM = 4096
BN = 640


def _fused_kernel(ct_ref, x_ref, emb_ref, w_ref, b_ref, o_ref):
    ct = ct_ref[0]
    x = x_ref[...]
    emb_table = emb_ref[...]

    types = jax.lax.broadcasted_iota(jnp.int32, (BM, N_TYPES), 1)
    onehot = (ct.reshape(BM, 1) == types).astype(jnp.float32)
    emb = jax.lax.dot_general(
        onehot, emb_table, (((1,), (0,)), ((), ())),
        preferred_element_type=jnp.float32)

    y = jnp.maximum(x * emb, 0.0).astype(jnp.bfloat16)

    out = jax.lax.dot_general(
        y, w_ref[...].astype(jnp.bfloat16), (((1,), (1,)), ((), ())),
        preferred_element_type=jnp.float32)
    o_ref[...] = out + b_ref[...]


def kernel(x, cell_types, emb_table, fc_w, fc_b):
    nb = BATCH // BM
    nn = pl.cdiv(N_CLASSES, BN)
    ct3 = cell_types.astype(jnp.int32).reshape(nb, 1, BM)
    b2 = fc_b.reshape(1, N_CLASSES)
    return pl.pallas_call(
        _fused_kernel,
        grid=(nb, nn),
        in_specs=[
            pl.BlockSpec((1, 1, BM), lambda i, j: (i, 0, 0)),
            pl.BlockSpec((BM, EMB_DIM), lambda i, j: (i, 0)),
            pl.BlockSpec((N_TYPES, EMB_DIM), lambda i, j: (0, 0)),
            pl.BlockSpec((BN, EMB_DIM), lambda i, j: (j, 0)),
            pl.BlockSpec((1, BN), lambda i, j: (0, j)),
        ],
        out_specs=pl.BlockSpec((BM, BN), lambda i, j: (i, j)),
        out_shape=jax.ShapeDtypeStruct((BATCH, N_CLASSES), jnp.float32),
        compiler_params=pltpu.CompilerParams(
            dimension_semantics=("parallel", "arbitrary")),
    )(ct3, x, emb_table, fc_w, b2)


# speedup vs baseline: 1.1010x; 1.1010x over previous
import jax
import jax.numpy as jnp
from jax.experimental import pallas as pl
from jax.experimental.pallas import tpu as pltpu

EMB_DIM = 512
N_CLASSES = 1139
N_TYPES = 4
BATCH = 16384

BM = 2048


def _fused_kernel(ct_ref, x_ref, emb_ref, w_ref, b_ref, o_ref):
    ct = ct_ref[0]
    x = x_ref[...]
    emb_table = emb_ref[...]

    types = jax.lax.broadcasted_iota(jnp.int32, (BM, N_TYPES), 1)
    onehot = (ct.reshape(BM, 1) == types).astype(jnp.float32)
    emb = jax.lax.dot_general(
        onehot, emb_table, (((1,), (0,)), ((), ())),
        preferred_element_type=jnp.float32)

    y = jnp.maximum(x * emb, 0.0).astype(jnp.bfloat16)

    out = jax.lax.dot_general(
        y, w_ref[...], (((1,), (1,)), ((), ())),
        preferred_element_type=jnp.float32)
    o_ref[...] = out + b_ref[...]


def kernel(x, cell_types, emb_table, fc_w, fc_b):
    nb = BATCH // BM
    ct3 = cell_types.astype(jnp.int32).reshape(nb, 1, BM)
    w_bf16 = fc_w.astype(jnp.bfloat16)
    b2 = fc_b.reshape(1, N_CLASSES)
    return pl.pallas_call(
        _fused_kernel,
        grid=(nb,),
        in_specs=[
            pl.BlockSpec((1, 1, BM), lambda i: (i, 0, 0)),
            pl.BlockSpec((BM, EMB_DIM), lambda i: (i, 0)),
            pl.BlockSpec((N_TYPES, EMB_DIM), lambda i: (0, 0)),
            pl.BlockSpec((N_CLASSES, EMB_DIM), lambda i: (0, 0)),
            pl.BlockSpec((1, N_CLASSES), lambda i: (0, 0)),
        ],
        out_specs=pl.BlockSpec((BM, N_CLASSES), lambda i: (i, 0)),
        out_shape=jax.ShapeDtypeStruct((BATCH, N_CLASSES), jnp.float32),
        compiler_params=pltpu.CompilerParams(
            dimension_semantics=("parallel",)),
    )(ct3, x, emb_table, w_bf16, b2)


# speedup vs baseline: 1.1258x; 1.0225x over previous
import jax
import jax.numpy as jnp
from jax.experimental import pallas as pl
from jax.experimental.pallas import tpu as pltpu

EMB_DIM = 512
N_CLASSES = 1139
N_TYPES = 4
BATCH = 16384

BM = 2048


def _fused_kernel(ct_ref, x_ref, emb_ref, w_ref, b_ref, o_ref):
    ct = ct_ref[0]
    x = x_ref[...]
    emb_table = emb_ref[...]

    types = jax.lax.broadcasted_iota(jnp.int32, (BM, N_TYPES), 1)
    onehot = (ct.reshape(BM, 1) == types).astype(jnp.float32)
    emb = jax.lax.dot_general(
        onehot, emb_table, (((1,), (0,)), ((), ())),
        preferred_element_type=jnp.float32)

    y = jnp.maximum(x * emb, 0.0).astype(jnp.bfloat16)

    out = jax.lax.dot_general(
        y, w_ref[...].astype(jnp.bfloat16), (((1,), (1,)), ((), ())),
        preferred_element_type=jnp.float32)
    o_ref[...] = out + b_ref[...]


def kernel(x, cell_types, emb_table, fc_w, fc_b):
    nb = BATCH // BM
    ct3 = cell_types.astype(jnp.int32).reshape(nb, 1, BM)
    b2 = fc_b.reshape(1, N_CLASSES)
    return pl.pallas_call(
        _fused_kernel,
        grid=(nb,),
        in_specs=[
            pl.BlockSpec((1, 1, BM), lambda i: (i, 0, 0)),
            pl.BlockSpec((BM, EMB_DIM), lambda i: (i, 0)),
            pl.BlockSpec((N_TYPES, EMB_DIM), lambda i: (0, 0)),
            pl.BlockSpec((N_CLASSES, EMB_DIM), lambda i: (0, 0)),
            pl.BlockSpec((1, N_CLASSES), lambda i: (0, 0)),
        ],
        out_specs=pl.BlockSpec((BM, N_CLASSES), lambda i: (i, 0)),
        out_shape=jax.ShapeDtypeStruct((BATCH, N_CLASSES), jnp.float32),
        compiler_params=pltpu.CompilerParams(
            dimension_semantics=("parallel",)),
    )(ct3, x, emb_table, fc_w, b2)
